# baseline (speedup 1.0000x reference)
import functools

import jax
import jax.numpy as jnp
from jax import lax
from jax.experimental import pallas as pl
from jax.experimental.pallas import tpu as pltpu
from jax.experimental.pallas import tpu_sc as plsc

NC = 2
NS = 16
NW = NC * NS
CHUNK = 320
SPLITS = (128, 128, 64)
TCB = 512


def _widen_body(in_ref, out_ref):
    x = in_ref[...]
    eye = jnp.eye(x.shape[0], dtype=x.dtype)
    t = lax.dot_general(x, eye, (((0,), (0,)), ((), ())))
    out_ref[...] = jnp.concatenate([t, t], axis=1)


@functools.lru_cache(maxsize=None)
def _widen_tc_call(dim, v):
    grid = (v + TCB - 1) // TCB
    return pl.pallas_call(
        _widen_body,
        grid=(grid,),
        in_specs=[pl.BlockSpec((dim, TCB), lambda j: (0, j))],
        out_specs=pl.BlockSpec((TCB, 2 * dim), lambda j: (j, 0)),
        out_shape=jax.ShapeDtypeStruct((v, 2 * dim), jnp.float32),
    )


@functools.lru_cache(maxsize=None)
def _gather_call(n_rows, dimp):
    rows_per_w = n_rows // NW
    steps = rows_per_w // CHUNK
    pairs = steps // 2

    mesh = plsc.VectorSubcoreMesh(core_axis_name="c", subcore_axis_name="s")

    @functools.partial(
        pl.kernel,
        mesh=mesh,
        out_type=jax.ShapeDtypeStruct((n_rows, dimp), jnp.float32),
        scratch_types=[
            pltpu.VMEM((CHUNK,), jnp.int32),
            pltpu.VMEM((CHUNK,), jnp.int32),
            pltpu.VMEM((CHUNK, dimp), jnp.float32),
            pltpu.VMEM((CHUNK, dimp), jnp.float32),
            pltpu.SemaphoreType.DMA,
            pltpu.SemaphoreType.DMA,
            pltpu.SemaphoreType.DMA,
        ],
    )
    def k(table_hbm, idx_hbm, out_hbm, idx0, idx1, buf0, buf1, g0s, g1s, wsem):
        wid = lax.axis_index("s") * NC + lax.axis_index("c")
        row0 = wid * rows_per_w

        def fire(c, idx_v, buf, sem):
            pltpu.sync_copy(idx_hbm.at[pl.ds(row0 + c * CHUNK, CHUNK)], idx_v)
            off = 0
            for n in SPLITS:
                pltpu.async_copy(
                    table_hbm.at[idx_v.at[pl.ds(off, n)]],
                    buf.at[pl.ds(off, n)],
                    sem,
                )
                off += n

        def drain_gathers(buf, sem):
            off = 0
            for n in SPLITS:
                pltpu.make_async_copy(
                    table_hbm.at[pl.ds(0, n)],
                    buf.at[pl.ds(off, n)],
                    sem,
                ).wait()
                off += n

        def writeback(c, buf):
            return pltpu.async_copy(
                buf, out_hbm.at[pl.ds(row0 + c * CHUNK, CHUNK)], wsem
            )

        def drain_writebacks():
            pltpu.make_async_copy(
                buf0, out_hbm.at[pl.ds(row0, CHUNK)], wsem
            ).wait()
            pltpu.make_async_copy(
                buf1, out_hbm.at[pl.ds(row0, CHUNK)], wsem
            ).wait()

        def body(t, carry):
            c0 = 2 * t
            c1 = c0 + 1

            @pl.when(t > 0)
            def _():
                drain_writebacks()

            fire(c0, idx0, buf0, g0s)
            fire(c1, idx1, buf1, g1s)
            drain_gathers(buf0, g0s)
            writeback(c0, buf0)
            drain_gathers(buf1, g1s)
            writeback(c1, buf1)
            return carry

        lax.fori_loop(0, pairs, body, 0)
        drain_writebacks()

    return k


def kernel(x, table):
    b, l = x.shape
    n = b * l
    v, dim = table.shape
    table_w = _widen_tc_call(dim, v)(table.T)
    idx_flat = x.reshape(n).astype(jnp.int32)
    out_w = _gather_call(n, 2 * dim)(table_w, idx_flat)
    return out_w[:, :dim].reshape(b, l, dim)


# speedup vs baseline: 1.7998x; 1.7998x over previous
import functools

import jax
import jax.numpy as jnp
from jax import lax
from jax.experimental import pallas as pl
from jax.experimental.pallas import tpu as pltpu
from jax.experimental.pallas import tpu_sc as plsc

NC = 2
NS = 16
NW = NC * NS
CHUNK = 320
SPLITS = (128, 128, 64)


@functools.lru_cache(maxsize=None)
def _gather_call(n_rows, dimp):
    rows_per_w = n_rows // NW
    steps = rows_per_w // CHUNK
    pairs = steps // 2

    mesh = plsc.VectorSubcoreMesh(core_axis_name="c", subcore_axis_name="s")

    @functools.partial(
        pl.kernel,
        mesh=mesh,
        out_type=jax.ShapeDtypeStruct((n_rows, dimp), jnp.float32),
        scratch_types=[
            pltpu.VMEM((rows_per_w,), jnp.int32),
            pltpu.VMEM((CHUNK, dimp), jnp.float32),
            pltpu.VMEM((CHUNK, dimp), jnp.float32),
            pltpu.SemaphoreType.DMA,
            pltpu.SemaphoreType.DMA,
            pltpu.SemaphoreType.DMA,
        ],
    )
    def k(table_hbm, idx_hbm, out_hbm, idx_all, buf0, buf1, g0s, g1s, wsem):
        wid = lax.axis_index("s") * NC + lax.axis_index("c")
        row0 = wid * rows_per_w
        pltpu.sync_copy(idx_hbm.at[pl.ds(row0, rows_per_w)], idx_all)

        def fire(c, buf, sem):
            off = 0
            for n in SPLITS:
                pltpu.async_copy(
                    table_hbm.at[idx_all.at[pl.ds(c * CHUNK + off, n)]],
                    buf.at[pl.ds(off, n)],
                    sem,
                )
                off += n

        def drain_gathers(buf, sem):
            off = 0
            for n in SPLITS:
                pltpu.make_async_copy(
                    table_hbm.at[pl.ds(0, n)],
                    buf.at[pl.ds(off, n)],
                    sem,
                ).wait()
                off += n

        def writeback(c, buf):
            return pltpu.async_copy(
                buf, out_hbm.at[pl.ds(row0 + c * CHUNK, CHUNK)], wsem
            )

        def drain_writebacks():
            pltpu.make_async_copy(
                buf0, out_hbm.at[pl.ds(row0, CHUNK)], wsem
            ).wait()
            pltpu.make_async_copy(
                buf1, out_hbm.at[pl.ds(row0, CHUNK)], wsem
            ).wait()

        def body(t, carry):
            c0 = 2 * t
            c1 = c0 + 1

            @pl.when(t > 0)
            def _():
                drain_writebacks()

            fire(c0, buf0, g0s)
            fire(c1, buf1, g1s)
            drain_gathers(buf0, g0s)
            writeback(c0, buf0)
            drain_gathers(buf1, g1s)
            writeback(c1, buf1)
            return carry

        lax.fori_loop(0, pairs, body, 0)
        drain_writebacks()

    return k


def kernel(x, table):
    b, l = x.shape
    n = b * l
    v, dim = table.shape
    table_w = jnp.pad(table, ((0, 0), (0, 64)))
    idx_flat = x.reshape(n).astype(jnp.int32)
    out_w = _gather_call(n, 2 * dim)(table_w, idx_flat)
    return out_w[:, :dim].reshape(b, l, dim)


# speedup vs baseline: 1.8032x; 1.0019x over previous
import functools

import jax
import jax.numpy as jnp
from jax import lax
from jax.experimental import pallas as pl
from jax.experimental.pallas import tpu as pltpu
from jax.experimental.pallas import tpu_sc as plsc

NC = 2
NS = 16
NW = NC * NS
CHUNK = 320
SPLITS = (128, 128, 64)


@functools.lru_cache(maxsize=None)
def _gather_call(n_rows, dimp):
    rows_per_w = n_rows // NW
    steps = rows_per_w // CHUNK
    pairs = steps // 2

    mesh = plsc.VectorSubcoreMesh(core_axis_name="c", subcore_axis_name="s")

    @functools.partial(
        pl.kernel,
        mesh=mesh,
        out_type=jax.ShapeDtypeStruct((n_rows, dimp), jnp.float32),
        scratch_types=[
            pltpu.VMEM((rows_per_w,), jnp.int32),
            pltpu.VMEM((CHUNK, dimp), jnp.float32),
            pltpu.VMEM((CHUNK, dimp), jnp.float32),
            pltpu.SemaphoreType.DMA,
            pltpu.SemaphoreType.DMA,
            pltpu.SemaphoreType.DMA,
        ],
    )
    def k(table_hbm, idx_hbm, out_hbm, idx_all, buf0, buf1, g0s, g1s, wsem):
        wid = lax.axis_index("s") * NC + lax.axis_index("c")
        row0 = wid * rows_per_w
        pltpu.sync_copy(idx_hbm.at[pl.ds(row0, rows_per_w)], idx_all)

        def fire(c, buf, sem):
            off = 0
            for n in SPLITS:
                pltpu.async_copy(
                    table_hbm.at[idx_all.at[pl.ds(c * CHUNK + off, n)]],
                    buf.at[pl.ds(off, n)],
                    sem,
                )
                off += n

        def drain_gathers(buf, sem):
            off = 0
            for n in SPLITS:
                pltpu.make_async_copy(
                    table_hbm.at[pl.ds(0, n)],
                    buf.at[pl.ds(off, n)],
                    sem,
                ).wait()
                off += n

        def writeback(c, buf):
            return pltpu.async_copy(
                buf, out_hbm.at[pl.ds(row0 + c * CHUNK, CHUNK)], wsem
            )

        def drain_writebacks():
            pltpu.make_async_copy(
                buf0, out_hbm.at[pl.ds(row0, CHUNK)], wsem
            ).wait()
            pltpu.make_async_copy(
                buf1, out_hbm.at[pl.ds(row0, CHUNK)], wsem
            ).wait()

        def body(t, carry):
            c0 = 2 * t
            c1 = c0 + 1

            @pl.when(t > 0)
            def _():
                drain_writebacks()

            fire(c0, buf0, g0s)
            fire(c1, buf1, g1s)
            drain_gathers(buf0, g0s)
            writeback(c0, buf0)
            drain_gathers(buf1, g1s)
            writeback(c1, buf1)
            return carry

        lax.fori_loop(0, pairs, body, 0)
        drain_writebacks()

    return k


def kernel(x, table):
    b, l = x.shape
    n = b * l
    v, dim = table.shape
    table_w = jnp.pad(table, ((0, 0), (0, dim)))
    idx_flat = x.reshape(n).astype(jnp.int32)
    out_w = _gather_call(n, 2 * dim)(table_w, idx_flat)
    return out_w[:, :dim].reshape(b, l, dim)
